# Initial kernel scaffold; baseline (speedup 1.0000x reference)
import functools

import jax
import jax.numpy as jnp
from jax import lax
from jax.experimental import pallas as pl
from jax.experimental.pallas import tpu as pltpu
from jax.experimental.pallas import tpu_sc as plsc

D = 64
B = 16384 * 50
NC, NS = 2, 16
NW = NC * NS
SUB = 128
N_SUB = 4
CHUNK = SUB * N_SUB
B_PER_W = B // NW
N_CHUNKS = B_PER_W // CHUNK


def _make_kernel():
  mesh = plsc.VectorSubcoreMesh(core_axis_name="c", subcore_axis_name="s")

  @functools.partial(
      pl.kernel,
      mesh=mesh,
      out_type=jax.ShapeDtypeStruct((B // SUB, SUB, D), jnp.float32),
      scratch_types=[
          pltpu.VMEM((N_SUB, SUB), jnp.int32),
          pltpu.VMEM((N_SUB, SUB, D), jnp.float32),
          pltpu.SemaphoreType.DMA,
      ],
  )
  def k(idx_hbm, table_hbm, out_hbm, idx_v, rows_v, sem):
    wid = lax.axis_index("s") * NC + lax.axis_index("c")
    row_base = wid * (B_PER_W // SUB)

    def body(i, carry):
      row_off = row_base + i * N_SUB
      pltpu.sync_copy(idx_hbm.at[pl.ds(row_off, N_SUB)], idx_v)
      copies = [
          pltpu.async_copy(table_hbm.at[idx_v.at[j]], rows_v.at[j], sem)
          for j in range(N_SUB)
      ]
      for c in copies:
        c.wait()
      pltpu.sync_copy(rows_v, out_hbm.at[pl.ds(row_off, N_SUB)])
      return carry

    lax.fori_loop(0, N_CHUNKS, body, 0)

  return k


_gather_call = _make_kernel()


@jax.jit
def kernel(inputs, weight):
  idx = inputs.reshape(-1).astype(jnp.int32).reshape(B // SUB, SUB)
  out = _gather_call(idx, weight)
  return out.reshape(inputs.shape + (weight.shape[1],))


# speedup vs baseline: 1.7983x; 1.7983x over previous
import functools

import jax
import jax.numpy as jnp
from jax import lax
from jax.experimental import pallas as pl
from jax.experimental.pallas import tpu as pltpu
from jax.experimental.pallas import tpu_sc as plsc

D = 64
B = 16384 * 50
NC, NS = 2, 16
NW = NC * NS
SUB = 128
N_SUB = 4
CHUNK = SUB * N_SUB
B_PER_W = B // NW
N_CHUNKS = B_PER_W // CHUNK


def _make_kernel():
  mesh = plsc.VectorSubcoreMesh(core_axis_name="c", subcore_axis_name="s")

  @functools.partial(
      pl.kernel,
      mesh=mesh,
      compiler_params=pltpu.CompilerParams(use_tc_tiling_on_sc=False),
      out_type=jax.ShapeDtypeStruct((B // SUB, SUB, D), jnp.float32),
      scratch_types=[
          pltpu.VMEM((N_SUB, SUB), jnp.int32),
          pltpu.VMEM((N_SUB, SUB, D), jnp.float32),
          pltpu.SemaphoreType.DMA,
      ],
  )
  def k(idx_hbm, table_hbm, out_hbm, idx_v, rows_v, sem):
    wid = lax.axis_index("s") * NC + lax.axis_index("c")
    row_base = wid * (B_PER_W // SUB)

    def body(i, carry):
      row_off = row_base + i * N_SUB
      pltpu.sync_copy(idx_hbm.at[pl.ds(row_off, N_SUB)], idx_v)
      copies = [
          pltpu.async_copy(table_hbm.at[idx_v.at[j]], rows_v.at[j], sem)
          for j in range(N_SUB)
      ]
      for c in copies:
        c.wait()
      pltpu.sync_copy(rows_v, out_hbm.at[pl.ds(row_off, N_SUB)])
      return carry

    lax.fori_loop(0, N_CHUNKS, body, 0)

  return k


_gather_call = _make_kernel()


@jax.jit
def kernel(inputs, weight):
  idx = inputs.reshape(-1).astype(jnp.int32).reshape(B // SUB, SUB)
  out = _gather_call(idx, weight)
  return out.reshape(inputs.shape + (weight.shape[1],))


# speedup vs baseline: 1.8658x; 1.0375x over previous
import functools

import jax
import jax.numpy as jnp
from jax import lax
from jax.experimental import pallas as pl
from jax.experimental.pallas import tpu as pltpu
from jax.experimental.pallas import tpu_sc as plsc

D = 64
B = 16384 * 50
NC, NS = 2, 16
NW = NC * NS
SUB = 128
N_SUB = 4
CHUNK = SUB * N_SUB
B_PER_W = B // NW
IDX_ROWS = B_PER_W // SUB
N_CHUNKS = B_PER_W // CHUNK
T = N_CHUNKS // 2


def _make_kernel():
  mesh = plsc.VectorSubcoreMesh(core_axis_name="c", subcore_axis_name="s")

  @functools.partial(
      pl.kernel,
      mesh=mesh,
      compiler_params=pltpu.CompilerParams(use_tc_tiling_on_sc=False),
      out_type=jax.ShapeDtypeStruct((B // SUB, SUB, D), jnp.float32),
      scratch_types=[
          pltpu.VMEM((IDX_ROWS, SUB), jnp.int32),
          pltpu.VMEM((2, N_SUB, SUB, D), jnp.float32),
          pltpu.SemaphoreType.DMA,
          pltpu.SemaphoreType.DMA,
          pltpu.SemaphoreType.DMA,
          pltpu.SemaphoreType.DMA,
      ],
  )
  def k(idx_hbm, table_hbm, out_hbm, idx_v, rows_v, sg0, sg1, sw0, sw1):
    wid = lax.axis_index("s") * NC + lax.axis_index("c")
    row_base = wid * IDX_ROWS
    pltpu.sync_copy(idx_hbm.at[pl.ds(row_base, IDX_ROWS)], idx_v)
    sg = (sg0, sg1)
    sw = (sw0, sw1)

    def fire(buf, c):
      for j in range(N_SUB):
        pltpu.async_copy(table_hbm.at[idx_v.at[c * N_SUB + j]],
                         rows_v.at[buf, j], sg[buf])

    def wait_gathers(buf):
      for j in range(N_SUB):
        pltpu.make_async_copy(table_hbm.at[idx_v.at[j]],
                              rows_v.at[buf, j], sg[buf]).wait()

    def write(buf, c):
      return pltpu.async_copy(
          rows_v.at[buf],
          out_hbm.at[pl.ds(row_base + c * N_SUB, N_SUB)], sw[buf])

    fire(0, 0)
    fire(1, 1)

    def body(t, carry):
      c0 = 2 * t
      c1 = c0 + 1
      wait_gathers(0)
      w0 = write(0, c0)
      wait_gathers(1)
      w1 = write(1, c1)
      w0.wait()

      @pl.when(t < T - 1)
      def _():
        fire(0, c0 + 2)

      w1.wait()

      @pl.when(t < T - 1)
      def _():
        fire(1, c1 + 2)

      return carry

    lax.fori_loop(0, T, body, 0)

  return k


_gather_call = _make_kernel()


@jax.jit
def kernel(inputs, weight):
  idx = inputs.reshape(-1).astype(jnp.int32).reshape(B // SUB, SUB)
  out = _gather_call(idx, weight)
  return out.reshape(inputs.shape + (weight.shape[1],))


# speedup vs baseline: 1.9670x; 1.0542x over previous
import functools

import jax
import jax.numpy as jnp
from jax import lax
from jax.experimental import pallas as pl
from jax.experimental.pallas import tpu as pltpu
from jax.experimental.pallas import tpu_sc as plsc

D = 64
B = 16384 * 50
NC, NS = 2, 16
NW = NC * NS
SUB = 128
N_SUB = 4
CHUNK = SUB * N_SUB
B_PER_W = B // NW
IDX_ROWS = B_PER_W // SUB
N_CHUNKS = B_PER_W // CHUNK
T = N_CHUNKS // 2


def _make_kernel():
  mesh = plsc.VectorSubcoreMesh(core_axis_name="c", subcore_axis_name="s")

  @functools.partial(
      pl.kernel,
      mesh=mesh,
      compiler_params=pltpu.CompilerParams(use_tc_tiling_on_sc=False),
      out_type=jax.ShapeDtypeStruct((B // SUB, SUB, D), jnp.float32),
      scratch_types=[
          pltpu.VMEM((IDX_ROWS, SUB), jnp.int32),
          pltpu.VMEM((2, N_SUB, SUB, D), jnp.float32),
          pltpu.SemaphoreType.DMA,
          pltpu.SemaphoreType.DMA,
          pltpu.SemaphoreType.DMA,
          pltpu.SemaphoreType.DMA,
      ],
  )
  def k(idx_hbm, table_hbm, out_hbm, idx_v, rows_v, sg0, sg1, sw0, sw1):
    wid = lax.axis_index("s") * NC + lax.axis_index("c")
    row_base = wid * IDX_ROWS
    pltpu.sync_copy(idx_hbm.at[pl.ds(row_base, IDX_ROWS)], idx_v)
    sg = (sg0, sg1)
    sw = (sw0, sw1)

    def fire(buf, c):
      for j in range(N_SUB):
        pltpu.async_copy(table_hbm.at[idx_v.at[c * N_SUB + j]],
                         rows_v.at[buf, j], sg[buf])

    def wait_gathers(buf):
      for j in range(N_SUB):
        pltpu.make_async_copy(table_hbm.at[idx_v.at[j]],
                              rows_v.at[buf, j], sg[buf]).wait()

    def write(buf, c):
      return pltpu.async_copy(
          rows_v.at[buf],
          out_hbm.at[pl.ds(row_base + c * N_SUB, N_SUB)], sw[buf])

    fire(0, 0)
    fire(1, 1)

    def body(t, carry):
      c0 = 2 * t
      c1 = c0 + 1
      wait_gathers(0)
      w0 = write(0, c0)
      wait_gathers(1)
      w1 = write(1, c1)
      w0.wait()

      @pl.when(t < T - 1)
      def _():
        fire(0, c0 + 2)

      w1.wait()

      @pl.when(t < T - 1)
      def _():
        fire(1, c1 + 2)

      return carry

    lax.fori_loop(0, T, body, 0)

  return k


_gather_call = _make_kernel()


@jax.jit
def kernel(inputs, weight):
  wlin = jnp.pad(weight, ((0, 0), (0, D))).reshape(2 * weight.shape[0], D)
  idx = (inputs.reshape(-1).astype(jnp.int32) * 2).reshape(B // SUB, SUB)
  out = _gather_call(idx, wlin)
  return out.reshape(inputs.shape + (weight.shape[1],))
